# baseline (speedup 1.0000x reference)
import functools

import jax
import jax.numpy as jnp
import numpy as np
from jax import lax

_PREC = lax.Precision.HIGHEST
from jax.experimental import pallas as pl
from jax.experimental.pallas import tpu as pltpu
from jax.experimental.pallas import tpu_sc as plsc

H = 128
R = 64
NP_ = 9000
NL = 1000
N = NP_ + NL
NPAD = 10240
E = 160000
B = 16
CUTOFF = 8.0
HEADS = 8
HD = H // HEADS

NC = 2
NS = 16
NW = NC * NS
CH = 128
EP = 163840
EPW = EP // NW
NCH = EPW // CH
ROWS_PER_TILE = NPAD // NS

_Z_TABLE = np.array([6, 8, 7, 16, 15, 1, 9, 17, 35, 53, 0], dtype=np.int32)


def _embed_body(xf_ref, et_ref, out_ref):
    xb = xf_ref[...]
    m = jnp.max(xb, axis=1, keepdims=True)
    iot = lax.broadcasted_iota(jnp.int32, xb.shape, 1)
    jmin = jnp.min(jnp.where(xb >= m, iot, H), axis=1, keepdims=True)
    onehot = (iot == jmin).astype(jnp.float32)
    out_ref[...] = jnp.dot(onehot, et_ref[...],
                           preferred_element_type=jnp.float32, precision=_PREC)


def _embed_lookup(xfeat_pad, embed_t_pad):
    bn = 1024
    return pl.pallas_call(
        _embed_body,
        grid=(NPAD // bn,),
        in_specs=[
            pl.BlockSpec((bn, H), lambda i: (i, 0)),
            pl.BlockSpec((H, H), lambda i: (0, 0)),
        ],
        out_specs=pl.BlockSpec((bn, H), lambda i: (i, 0)),
        out_shape=jax.ShapeDtypeStruct((NPAD, H), jnp.float32),
    )(xfeat_pad, embed_t_pad)


def _rbf_body(ps_ref, pd_ref, means_ref, rbf_ref, c_ref, *, block):
    pid = pl.program_id(0)
    rij = pd_ref[...] - ps_ref[...]
    d2 = jnp.sum(rij * rij, axis=1, keepdims=True)
    d = jnp.sqrt(d2 + 1e-12)
    beta = np.float32((2.0 / R * (1.0 - np.exp(-CUTOFF))) ** -2)
    rbf_ref[...] = jnp.exp(-beta * (jnp.exp(-d) - means_ref[...]) ** 2)
    row = pid * block + lax.broadcasted_iota(jnp.int32, d.shape, 0)
    cc = 0.5 * (jnp.cos(np.float32(np.pi) * jnp.clip(d, 0.0, CUTOFF)
                        / CUTOFF) + 1.0)
    cc = cc * (d < CUTOFF).astype(jnp.float32)
    c_ref[...] = jnp.where(row < E, cc, 0.0)


def _rbf_cutoff(xs0, xd0, means_row):
    be = 4096
    return pl.pallas_call(
        functools.partial(_rbf_body, block=be),
        grid=(EP // be,),
        in_specs=[
            pl.BlockSpec((be, 128), lambda i: (i, 1)),
            pl.BlockSpec((be, 128), lambda i: (i, 1)),
            pl.BlockSpec((1, R), lambda i: (0, 0)),
        ],
        out_specs=[
            pl.BlockSpec((be, R), lambda i: (i, 0)),
            pl.BlockSpec((be, 1), lambda i: (i, 0)),
        ],
        out_shape=[
            jax.ShapeDtypeStruct((EP, R), jnp.float32),
            jax.ShapeDtypeStruct((EP, 1), jnp.float32),
        ],
    )(xs0, xd0, means_row)


def _silu(t):
    return t * (1.0 / (1.0 + jnp.exp(-t)))


def _wo_body(x_ref, d0_ref, d1_ref, wo_ref, out_ref):
    dx = d0_ref[...] + d1_ref[...]
    out_ref[...] = x_ref[...] + _silu(
        jnp.dot(dx, wo_ref[...], preferred_element_type=jnp.float32, precision=_PREC))


def _residual_update(x, dxp, wo):
    bn = 1024
    return pl.pallas_call(
        _wo_body,
        grid=(NPAD // bn,),
        in_specs=[
            pl.BlockSpec((bn, H), lambda i: (i, 0)),
            pl.BlockSpec((bn, H), lambda i: (i, 0)),
            pl.BlockSpec((bn, H), lambda i: (i + NPAD // bn, 0)),
            pl.BlockSpec((H, H), lambda i: (0, 0)),
        ],
        out_specs=pl.BlockSpec((bn, H), lambda i: (i, 0)),
        out_shape=jax.ShapeDtypeStruct((NPAD, H), jnp.float32),
    )(x, dxp, dxp, wo)


def _readout_body(x_ref, b_ref, w1_ref, w2_ref, y_ref):
    pid = pl.program_id(0)
    h = jnp.dot(_silu(jnp.dot(x_ref[...], w1_ref[...],
                              preferred_element_type=jnp.float32, precision=_PREC)),
                w2_ref[...], preferred_element_type=jnp.float32, precision=_PREC)
    iot = lax.broadcasted_iota(jnp.int32, (x_ref.shape[0], B), 1)
    onehot = (b_ref[...] == iot).astype(jnp.float32)
    yb = lax.dot_general(onehot, h, (((0,), (0,)), ((), ())),
                         preferred_element_type=jnp.float32, precision=_PREC)

    @pl.when(pid == 0)
    def _():
        y_ref[...] = jnp.zeros_like(y_ref)

    y_ref[...] += yb


def _readout(x, batch2d, w1, w2):
    bn = 1024
    return pl.pallas_call(
        _readout_body,
        grid=(NPAD // bn,),
        in_specs=[
            pl.BlockSpec((bn, H), lambda i: (i, 0)),
            pl.BlockSpec((bn, 1), lambda i: (i, 0)),
            pl.BlockSpec((H, H // 2), lambda i: (0, 0)),
            pl.BlockSpec((H // 2, 1), lambda i: (0, 0)),
        ],
        out_specs=pl.BlockSpec((B, 1), lambda i: (0, 0)),
        out_shape=jax.ShapeDtypeStruct((B, 1), jnp.float32),
    )(x, batch2d, w1, w2)


_SC_MESH = plsc.VectorSubcoreMesh(core_axis_name="c", subcore_axis_name="s")


def _x_gather_body(x_hbm, src_hbm, dst_hbm, xs_hbm, xd_hbm,
                   sidx, didx, xsb, xdb):
    c = lax.axis_index("c")
    s = lax.axis_index("s")
    wid = c * NS + s

    def chunk(ch, carry):
        base = wid * EPW + ch * CH
        pltpu.sync_copy(src_hbm.at[pl.ds(base, CH)], sidx)
        pltpu.sync_copy(dst_hbm.at[pl.ds(base, CH)], didx)
        pltpu.sync_copy(x_hbm.at[sidx], xsb)
        pltpu.sync_copy(x_hbm.at[didx], xdb)
        pltpu.sync_copy(xsb, xs_hbm.at[pl.ds(base, CH)])
        pltpu.sync_copy(xdb, xd_hbm.at[pl.ds(base, CH)])
        return carry

    lax.fori_loop(0, NCH, chunk, 0)


def _x_gather(x, srcp, dstp, w):
    f = pl.kernel(
        _x_gather_body,
        out_type=[jax.ShapeDtypeStruct((EP, w), jnp.float32)] * 2,
        mesh=_SC_MESH,
        scratch_types=[
            pltpu.VMEM((CH,), jnp.int32),
            pltpu.VMEM((CH,), jnp.int32),
            pltpu.VMEM((CH, w), jnp.float32),
            pltpu.VMEM((CH, w), jnp.float32),
        ],
    )
    return f(x, srcp, dstp)


def _attn_body(xd_ref, xs_ref, rbf_ref, c_ref, wq_ref, wk_ref, wv_ref,
               wdk_ref, wdv_ref, gh_ref, msg_ref):
    xd = xd_ref[...]
    xs = xs_ref[...]
    q = jnp.dot(xd, wq_ref[...], preferred_element_type=jnp.float32, precision=_PREC)
    k = jnp.dot(xs, wk_ref[...], preferred_element_type=jnp.float32, precision=_PREC)
    v = jnp.dot(xs, wv_ref[...], preferred_element_type=jnp.float32, precision=_PREC)
    rb = rbf_ref[...]
    dk = _silu(jnp.dot(rb, wdk_ref[...],
                       preferred_element_type=jnp.float32, precision=_PREC))
    dv = _silu(jnp.dot(rb, wdv_ref[...],
                       preferred_element_type=jnp.float32, precision=_PREC))
    p = q * k * dk
    gh = gh_ref[...]
    dots = jnp.dot(p, gh, preferred_element_type=jnp.float32, precision=_PREC)
    attn8 = _silu(dots)
    attn = lax.dot_general(attn8, gh, (((1,), (1,)), ((), ())),
                           preferred_element_type=jnp.float32, precision=_PREC)
    msg_ref[...] = v * dv * c_ref[...] * attn


def _attn_msg(xd, xs, rbf, c, wq, wk, wv, wdk, wdv, gh):
    be = 4096
    return pl.pallas_call(
        _attn_body,
        grid=(EP // be,),
        in_specs=[
            pl.BlockSpec((be, H), lambda i: (i, 0)),
            pl.BlockSpec((be, H), lambda i: (i, 0)),
            pl.BlockSpec((be, R), lambda i: (i, 0)),
            pl.BlockSpec((be, 1), lambda i: (i, 0)),
            pl.BlockSpec((H, H), lambda i: (0, 0)),
            pl.BlockSpec((H, H), lambda i: (0, 0)),
            pl.BlockSpec((H, H), lambda i: (0, 0)),
            pl.BlockSpec((R, H), lambda i: (0, 0)),
            pl.BlockSpec((R, H), lambda i: (0, 0)),
            pl.BlockSpec((H, HEADS), lambda i: (0, 0)),
        ],
        out_specs=pl.BlockSpec((be, H), lambda i: (i, 0)),
        out_shape=jax.ShapeDtypeStruct((EP, H), jnp.float32),
    )(xd, xs, rbf, c, wq, wk, wv, wdk, wdv, gh)


def _scatter_body(msg_hbm, dst_hbm, z_hbm, out_hbm, acc, didx, msgb):
    c = lax.axis_index("c")
    s = lax.axis_index("s")
    wid = c * NS + s
    row0 = s * ROWS_PER_TILE

    pltpu.sync_copy(z_hbm, acc.at[pl.ds(row0, ROWS_PER_TILE)])
    plsc.subcore_barrier()

    def chunk(ch, carry):
        base = wid * EPW + ch * CH
        pltpu.sync_copy(dst_hbm.at[pl.ds(base, CH)], didx)
        pltpu.sync_copy(msg_hbm.at[pl.ds(base, CH)], msgb)
        pltpu.sync_copy(msgb, acc.at[didx], add=True)
        return carry

    lax.fori_loop(0, NCH, chunk, 0)
    plsc.subcore_barrier()
    pltpu.sync_copy(acc.at[pl.ds(row0, ROWS_PER_TILE)],
                    out_hbm.at[pl.ds(c * NPAD + row0, ROWS_PER_TILE)])


def _edge_scatter(msg, dstp, zrows):
    f = pl.kernel(
        _scatter_body,
        out_type=jax.ShapeDtypeStruct((NC * NPAD, H), jnp.float32),
        mesh=_SC_MESH,
        scratch_types=[
            pltpu.VMEM_SHARED((NPAD, H), jnp.float32),
            pltpu.VMEM((CH,), jnp.int32),
            pltpu.VMEM((CH, H), jnp.float32),
        ],
    )
    return f(msg, dstp, zrows)


def kernel(protein_x, ligand_x, protein_pos, ligand_pos, protein_x_batch,
           ligand_x_batch, edge_index, embed, Wq, Wk, Wv, Wdk, Wdv, Wo,
           Wout1, Wout2):
    xfeat = jnp.concatenate([protein_x, ligand_x], axis=0)
    xfeat_pad = jnp.pad(xfeat, ((0, NPAD - N), (0, H - xfeat.shape[1])),
                        constant_values=-1e30)
    pos = jnp.concatenate([protein_pos, ligand_pos], axis=0)
    pospad = jnp.pad(pos, ((0, NPAD - N), (0, 128 - pos.shape[1])))
    batch2d = jnp.pad(
        jnp.concatenate([protein_x_batch, ligand_x_batch], axis=0),
        (0, NPAD - N), constant_values=B)[:, None]
    srcp = jnp.pad(edge_index[0], (0, EP - E))
    dstp = jnp.pad(edge_index[1], (0, EP - E))
    embed_t = embed[jnp.asarray(_Z_TABLE)]
    embed_t_pad = jnp.pad(embed_t, ((0, H - embed_t.shape[0]), (0, 0)))
    means_row = jnp.linspace(np.exp(-CUTOFF), 1.0, R,
                             dtype=jnp.float32)[None, :]
    zrows = jnp.zeros((ROWS_PER_TILE, H), jnp.float32)
    gh = jnp.asarray(np.kron(np.eye(HEADS, dtype=np.float32),
                             np.ones((HD, 1), dtype=np.float32)))

    x = _embed_lookup(xfeat_pad, embed_t_pad)
    xpos = jnp.concatenate([x, pospad], axis=1)
    xs, xd = _x_gather(xpos, srcp, dstp, 2 * H)
    rbf, c = _rbf_cutoff(xs, xd, means_row)

    n_layers = Wq.shape[0]
    for l in range(n_layers):
        if l > 0:
            xs, xd = _x_gather(x, srcp, dstp, H)
        msg = _attn_msg(xd, xs, rbf, c, Wq[l], Wk[l], Wv[l],
                        Wdk[l], Wdv[l], gh)
        dxp = _edge_scatter(msg, dstp, zrows)
        x = _residual_update(x, dxp, Wo[l])

    return _readout(x, batch2d, Wout1, Wout2)


# speedup vs baseline: 1.0693x; 1.0693x over previous
import functools

import jax
import jax.numpy as jnp
import numpy as np
from jax import lax

_PREC = lax.Precision.HIGHEST
from jax.experimental import pallas as pl
from jax.experimental.pallas import tpu as pltpu
from jax.experimental.pallas import tpu_sc as plsc

H = 128
R = 64
NP_ = 9000
NL = 1000
N = NP_ + NL
NPAD = 10240
E = 160000
B = 16
CUTOFF = 8.0
HEADS = 8
HD = H // HEADS

NC = 2
NS = 16
NW = NC * NS
CH = 128
EP = 163840
EPW = EP // NW
NCH = EPW // CH
ROWS_PER_TILE = NPAD // NS

_Z_TABLE = np.array([6, 8, 7, 16, 15, 1, 9, 17, 35, 53, 0], dtype=np.int32)


def _embed_body(xf_ref, et_ref, out_ref):
    xb = xf_ref[...]
    m = jnp.max(xb, axis=1, keepdims=True)
    iot = lax.broadcasted_iota(jnp.int32, xb.shape, 1)
    jmin = jnp.min(jnp.where(xb >= m, iot, H), axis=1, keepdims=True)
    onehot = (iot == jmin).astype(jnp.float32)
    out_ref[...] = jnp.dot(onehot, et_ref[...],
                           preferred_element_type=jnp.float32, precision=_PREC)


def _embed_lookup(xfeat_pad, embed_t_pad):
    bn = 1024
    return pl.pallas_call(
        _embed_body,
        grid=(NPAD // bn,),
        in_specs=[
            pl.BlockSpec((bn, H), lambda i: (i, 0)),
            pl.BlockSpec((H, H), lambda i: (0, 0)),
        ],
        out_specs=pl.BlockSpec((bn, H), lambda i: (i, 0)),
        out_shape=jax.ShapeDtypeStruct((NPAD, H), jnp.float32),
    )(xfeat_pad, embed_t_pad)


def _rbf_body(ps_ref, pd_ref, means_ref, rbf_ref, c_ref, *, block):
    pid = pl.program_id(0)
    rij = pd_ref[...] - ps_ref[...]
    d2 = jnp.sum(rij * rij, axis=1, keepdims=True)
    d = jnp.sqrt(d2 + 1e-12)
    beta = np.float32((2.0 / R * (1.0 - np.exp(-CUTOFF))) ** -2)
    rbf_ref[...] = jnp.exp(-beta * (jnp.exp(-d) - means_ref[...]) ** 2)
    row = pid * block + lax.broadcasted_iota(jnp.int32, d.shape, 0)
    cc = 0.5 * (jnp.cos(np.float32(np.pi) * jnp.clip(d, 0.0, CUTOFF)
                        / CUTOFF) + 1.0)
    cc = cc * (d < CUTOFF).astype(jnp.float32)
    c_ref[...] = jnp.where(row < E, cc, 0.0)


def _rbf_cutoff(ps, pd, means_row):
    be = 4096
    return pl.pallas_call(
        functools.partial(_rbf_body, block=be),
        grid=(EP // be,),
        in_specs=[
            pl.BlockSpec((be, 128), lambda i: (i, 0)),
            pl.BlockSpec((be, 128), lambda i: (i, 0)),
            pl.BlockSpec((1, R), lambda i: (0, 0)),
        ],
        out_specs=[
            pl.BlockSpec((be, R), lambda i: (i, 0)),
            pl.BlockSpec((be, 1), lambda i: (i, 0)),
        ],
        out_shape=[
            jax.ShapeDtypeStruct((EP, R), jnp.float32),
            jax.ShapeDtypeStruct((EP, 1), jnp.float32),
        ],
    )(ps, pd, means_row)


def _silu(t):
    return t * (1.0 / (1.0 + jnp.exp(-t)))


def _wo_body(x_ref, d0_ref, d1_ref, wo_ref, out_ref):
    dx = d0_ref[...] + d1_ref[...]
    out_ref[...] = x_ref[...] + _silu(
        jnp.dot(dx, wo_ref[...], preferred_element_type=jnp.float32, precision=_PREC))


def _residual_update(x, dxp, wo):
    bn = 1024
    return pl.pallas_call(
        _wo_body,
        grid=(NPAD // bn,),
        in_specs=[
            pl.BlockSpec((bn, H), lambda i: (i, 0)),
            pl.BlockSpec((bn, H), lambda i: (i, 0)),
            pl.BlockSpec((bn, H), lambda i: (i + NPAD // bn, 0)),
            pl.BlockSpec((H, H), lambda i: (0, 0)),
        ],
        out_specs=pl.BlockSpec((bn, H), lambda i: (i, 0)),
        out_shape=jax.ShapeDtypeStruct((NPAD, H), jnp.float32),
    )(x, dxp, dxp, wo)


def _readout_body(x_ref, b_ref, w1_ref, w2_ref, y_ref):
    pid = pl.program_id(0)
    h = jnp.dot(_silu(jnp.dot(x_ref[...], w1_ref[...],
                              preferred_element_type=jnp.float32, precision=_PREC)),
                w2_ref[...], preferred_element_type=jnp.float32, precision=_PREC)
    iot = lax.broadcasted_iota(jnp.int32, (x_ref.shape[0], B), 1)
    onehot = (b_ref[...] == iot).astype(jnp.float32)
    yb = lax.dot_general(onehot, h, (((0,), (0,)), ((), ())),
                         preferred_element_type=jnp.float32, precision=_PREC)

    @pl.when(pid == 0)
    def _():
        y_ref[...] = jnp.zeros_like(y_ref)

    y_ref[...] += yb


def _readout(x, batch2d, w1, w2):
    bn = 1024
    return pl.pallas_call(
        _readout_body,
        grid=(NPAD // bn,),
        in_specs=[
            pl.BlockSpec((bn, H), lambda i: (i, 0)),
            pl.BlockSpec((bn, 1), lambda i: (i, 0)),
            pl.BlockSpec((H, H // 2), lambda i: (0, 0)),
            pl.BlockSpec((H // 2, 1), lambda i: (0, 0)),
        ],
        out_specs=pl.BlockSpec((B, 1), lambda i: (0, 0)),
        out_shape=jax.ShapeDtypeStruct((B, 1), jnp.float32),
    )(x, batch2d, w1, w2)


_SC_MESH = plsc.VectorSubcoreMesh(core_axis_name="c", subcore_axis_name="s")


def _qkv_body(x_ref, wq_ref, wk_ref, wv_ref, q_ref, kv_ref):
    xb = x_ref[...]
    q_ref[...] = jnp.dot(xb, wq_ref[...], preferred_element_type=jnp.float32, precision=_PREC)
    kv_ref[:, :H] = jnp.dot(xb, wk_ref[...], preferred_element_type=jnp.float32, precision=_PREC)
    kv_ref[:, H:] = jnp.dot(xb, wv_ref[...], preferred_element_type=jnp.float32, precision=_PREC)


def _qkv(x, wq, wk, wv):
    bn = 1024
    return pl.pallas_call(
        _qkv_body,
        grid=(NPAD // bn,),
        in_specs=[
            pl.BlockSpec((bn, H), lambda i: (i, 0)),
            pl.BlockSpec((H, H), lambda i: (0, 0)),
            pl.BlockSpec((H, H), lambda i: (0, 0)),
            pl.BlockSpec((H, H), lambda i: (0, 0)),
        ],
        out_specs=[
            pl.BlockSpec((bn, H), lambda i: (i, 0)),
            pl.BlockSpec((bn, 2 * H), lambda i: (i, 0)),
        ],
        out_shape=[
            jax.ShapeDtypeStruct((NPAD, H), jnp.float32),
            jax.ShapeDtypeStruct((NPAD, 2 * H), jnp.float32),
        ],
    )(x, wq, wk, wv)


def _pos_gather_body(pos_hbm, src_hbm, dst_hbm, ps_hbm, pd_hbm,
                     sidx, didx, psb, pdb):
    c = lax.axis_index("c")
    s = lax.axis_index("s")
    wid = c * NS + s
    pltpu.sync_copy(src_hbm.at[pl.ds(wid * EPW, EPW)], sidx)
    pltpu.sync_copy(dst_hbm.at[pl.ds(wid * EPW, EPW)], didx)

    def chunk(ch, carry):
        base = wid * EPW + ch * CH
        off = ch * CH
        pltpu.sync_copy(pos_hbm.at[sidx.at[pl.ds(off, CH)]], psb)
        pltpu.sync_copy(pos_hbm.at[didx.at[pl.ds(off, CH)]], pdb)
        pltpu.sync_copy(psb, ps_hbm.at[pl.ds(base, CH)])
        pltpu.sync_copy(pdb, pd_hbm.at[pl.ds(base, CH)])
        return carry

    lax.fori_loop(0, NCH, chunk, 0)


def _pos_gather(pospad, srcp, dstp):
    f = pl.kernel(
        _pos_gather_body,
        out_type=[jax.ShapeDtypeStruct((EP, 128), jnp.float32)] * 2,
        mesh=_SC_MESH,
        scratch_types=[
            pltpu.VMEM((EPW,), jnp.int32),
            pltpu.VMEM((EPW,), jnp.int32),
            pltpu.VMEM((CH, 128), jnp.float32),
            pltpu.VMEM((CH, 128), jnp.float32),
        ],
    )
    return f(pospad, srcp, dstp)


def _gather_body(q_hbm, kv_hbm, src_hbm, dst_hbm, qd_hbm, kvs_hbm,
                 sidx, didx, qd, kvs):
    c = lax.axis_index("c")
    s = lax.axis_index("s")
    wid = c * NS + s
    pltpu.sync_copy(src_hbm.at[pl.ds(wid * EPW, EPW)], sidx)
    pltpu.sync_copy(dst_hbm.at[pl.ds(wid * EPW, EPW)], didx)

    def chunk(ch, carry):
        base = wid * EPW + ch * CH
        off = ch * CH
        pltpu.sync_copy(q_hbm.at[didx.at[pl.ds(off, CH)]], qd)
        pltpu.sync_copy(kv_hbm.at[sidx.at[pl.ds(off, CH)]], kvs)
        pltpu.sync_copy(qd, qd_hbm.at[pl.ds(base, CH)])
        pltpu.sync_copy(kvs, kvs_hbm.at[pl.ds(base, CH)])
        return carry

    lax.fori_loop(0, NCH, chunk, 0)


def _edge_gather(q, kv, srcp, dstp):
    f = pl.kernel(
        _gather_body,
        out_type=[
            jax.ShapeDtypeStruct((EP, H), jnp.float32),
            jax.ShapeDtypeStruct((EP, 2 * H), jnp.float32),
        ],
        mesh=_SC_MESH,
        scratch_types=[
            pltpu.VMEM((EPW,), jnp.int32),
            pltpu.VMEM((EPW,), jnp.int32),
            pltpu.VMEM((CH, H), jnp.float32),
            pltpu.VMEM((CH, 2 * H), jnp.float32),
        ],
    )
    return f(q, kv, srcp, dstp)


def _attn_body(qd_ref, ks_ref, vs_ref, rbf_ref, c_ref, wdk_ref, wdv_ref,
               gh_ref, msg_ref):
    rb = rbf_ref[...]
    dk = _silu(jnp.dot(rb, wdk_ref[...],
                       preferred_element_type=jnp.float32, precision=_PREC))
    dv = _silu(jnp.dot(rb, wdv_ref[...],
                       preferred_element_type=jnp.float32, precision=_PREC))
    p = qd_ref[...] * ks_ref[...] * dk
    gh = gh_ref[...]
    dots = jnp.dot(p, gh, preferred_element_type=jnp.float32, precision=_PREC)
    attn8 = _silu(dots)
    attn = lax.dot_general(attn8, gh, (((1,), (1,)), ((), ())),
                           preferred_element_type=jnp.float32, precision=_PREC)
    msg_ref[...] = vs_ref[...] * dv * c_ref[...] * attn


def _attn_msg(qd, kvs, rbf, c, wdk, wdv, gh):
    be = 4096
    return pl.pallas_call(
        _attn_body,
        grid=(EP // be,),
        in_specs=[
            pl.BlockSpec((be, H), lambda i: (i, 0)),
            pl.BlockSpec((be, H), lambda i: (i, 0)),
            pl.BlockSpec((be, H), lambda i: (i, 1)),
            pl.BlockSpec((be, R), lambda i: (i, 0)),
            pl.BlockSpec((be, 1), lambda i: (i, 0)),
            pl.BlockSpec((R, H), lambda i: (0, 0)),
            pl.BlockSpec((R, H), lambda i: (0, 0)),
            pl.BlockSpec((H, HEADS), lambda i: (0, 0)),
        ],
        out_specs=pl.BlockSpec((be, H), lambda i: (i, 0)),
        out_shape=jax.ShapeDtypeStruct((EP, H), jnp.float32),
    )(qd, kvs, kvs, rbf, c, wdk, wdv, gh)


def _scatter_body(msg_hbm, dst_hbm, z_hbm, out_hbm, acc, didx, msgb):
    c = lax.axis_index("c")
    s = lax.axis_index("s")
    wid = c * NS + s
    row0 = s * ROWS_PER_TILE

    pltpu.sync_copy(dst_hbm.at[pl.ds(wid * EPW, EPW)], didx)
    pltpu.sync_copy(z_hbm, acc.at[pl.ds(row0, ROWS_PER_TILE)])
    plsc.subcore_barrier()

    def chunk(ch, carry):
        base = wid * EPW + ch * CH
        off = ch * CH
        pltpu.sync_copy(msg_hbm.at[pl.ds(base, CH)], msgb)
        pltpu.sync_copy(msgb, acc.at[didx.at[pl.ds(off, CH)]], add=True)
        return carry

    lax.fori_loop(0, NCH, chunk, 0)
    plsc.subcore_barrier()
    pltpu.sync_copy(acc.at[pl.ds(row0, ROWS_PER_TILE)],
                    out_hbm.at[pl.ds(c * NPAD + row0, ROWS_PER_TILE)])


def _edge_scatter(msg, dstp, zrows):
    f = pl.kernel(
        _scatter_body,
        out_type=jax.ShapeDtypeStruct((NC * NPAD, H), jnp.float32),
        mesh=_SC_MESH,
        scratch_types=[
            pltpu.VMEM_SHARED((NPAD, H), jnp.float32),
            pltpu.VMEM((EPW,), jnp.int32),
            pltpu.VMEM((CH, H), jnp.float32),
        ],
    )
    return f(msg, dstp, zrows)


def kernel(protein_x, ligand_x, protein_pos, ligand_pos, protein_x_batch,
           ligand_x_batch, edge_index, embed, Wq, Wk, Wv, Wdk, Wdv, Wo,
           Wout1, Wout2):
    xfeat = jnp.concatenate([protein_x, ligand_x], axis=0)
    xfeat_pad = jnp.pad(xfeat, ((0, NPAD - N), (0, H - xfeat.shape[1])),
                        constant_values=-1e30)
    pos = jnp.concatenate([protein_pos, ligand_pos], axis=0)
    pospad = jnp.pad(pos, ((0, NPAD - N), (0, 128 - pos.shape[1])))
    batch2d = jnp.pad(
        jnp.concatenate([protein_x_batch, ligand_x_batch], axis=0),
        (0, NPAD - N), constant_values=B)[:, None]
    srcp = jnp.pad(edge_index[0], (0, EP - E))
    dstp = jnp.pad(edge_index[1], (0, EP - E))
    embed_t = embed[jnp.asarray(_Z_TABLE)]
    embed_t_pad = jnp.pad(embed_t, ((0, H - embed_t.shape[0]), (0, 0)))
    means_row = jnp.linspace(np.exp(-CUTOFF), 1.0, R,
                             dtype=jnp.float32)[None, :]
    zrows = jnp.zeros((ROWS_PER_TILE, H), jnp.float32)
    gh = jnp.asarray(np.kron(np.eye(HEADS, dtype=np.float32),
                             np.ones((HD, 1), dtype=np.float32)))

    x = _embed_lookup(xfeat_pad, embed_t_pad)
    ps, pd = _pos_gather(pospad, srcp, dstp)
    rbf, c = _rbf_cutoff(ps, pd, means_row)

    n_layers = Wq.shape[0]
    for l in range(n_layers):
        q, kv = _qkv(x, Wq[l], Wk[l], Wv[l])
        qd, kvs = _edge_gather(q, kv, srcp, dstp)
        msg = _attn_msg(qd, kvs, rbf, c, Wdk[l], Wdv[l], gh)
        dxp = _edge_scatter(msg, dstp, zrows)
        x = _residual_update(x, dxp, Wo[l])

    return _readout(x, batch2d, Wout1, Wout2)
